# Initial kernel scaffold; baseline (speedup 1.0000x reference)
import jax
import jax.numpy as jnp
from jax.experimental import pallas as pl


def kernel(x, token_table, pos_table):
    raise NotImplementedError("write your pallas kernel here")


# speedup vs baseline: 1.3655x; 1.3655x over previous
import functools

import jax
import jax.numpy as jnp
from jax import lax
from jax.experimental import pallas as pl
from jax.experimental.pallas import tpu as pltpu
from jax.experimental.pallas import tpu_sc as plsc

D = 768
LANES = 16
NC, NS = 2, 16
NW = NC * NS
N = 8192
SEQ = 2048
PER_W = N // NW
C = 64
NCH = PER_W // C
GROUPS = D // LANES


@functools.partial(
    pl.kernel,
    out_type=jax.ShapeDtypeStruct((N, D), jnp.float32),
    mesh=plsc.VectorSubcoreMesh(
        core_axis_name="c", subcore_axis_name="s", num_cores=NC, num_subcores=NS
    ),
    scratch_types=[
        pltpu.VMEM((PER_W,), jnp.int32),
        pltpu.VMEM((PER_W,), jnp.int32),
        pltpu.VMEM((C, D), jnp.float32),
        pltpu.VMEM((C, D), jnp.float32),
        pltpu.SemaphoreType.DMA,
        pltpu.SemaphoreType.DMA,
    ],
)
def _embed_kernel(x_hbm, tok_hbm, pos_hbm, out_hbm,
                  idx_v, pidx_v, tok_buf, pos_buf, sem_t, sem_p):
    wid = lax.axis_index("s") * NC + lax.axis_index("c")
    base = wid * PER_W
    l0 = base % SEQ

    pltpu.sync_copy(x_hbm.at[pl.ds(base, PER_W)], idx_v)

    iota = lax.iota(jnp.int32, LANES)
    for j in range(PER_W // LANES):
        sl = pl.ds(j * LANES, LANES)
        tok = idx_v[sl]
        lvec = iota + (l0 + j * LANES + 1)
        pidx_v[sl] = jnp.where(tok > 0, lvec, 0)

    for ch in range(NCH):
        cp_t = pltpu.async_copy(
            tok_hbm.at[idx_v.at[pl.ds(ch * C, C)]], tok_buf, sem_t)
        cp_p = pltpu.async_copy(
            pos_hbm.at[pidx_v.at[pl.ds(ch * C, C)]], pos_buf, sem_p)
        cp_t.wait()
        cp_p.wait()

        def add_row(r, carry):
            for g in range(GROUPS):
                sl = pl.ds(g * LANES, LANES)
                tok_buf[r, sl] = tok_buf[r, sl] + pos_buf[r, sl]
            return carry

        lax.fori_loop(0, C, add_row, 0)

        pltpu.sync_copy(tok_buf, out_hbm.at[pl.ds(base + ch * C, C)])


def kernel(x, token_table, pos_table):
    B, L = x.shape
    out = _embed_kernel(x.reshape(-1), token_table, pos_table)
    return out.reshape(B, L, D)
